# baseline (speedup 1.0000x reference)
import jax
import jax.numpy as jnp
from jax.experimental import pallas as pl
from jax.experimental.pallas import tpu as pltpu

_HIDDEN = 1024
_NUM_EMB = 3
_SUB = 1024
_NBUF = 4
_STEP = _SUB * _NBUF


def _emb_body(x_ref, t_ref, o_ref, buf, sems):
    i = pl.program_id(0)
    r1 = t_ref[1, :][None, :]
    r2 = t_ref[2, :][None, :]
    zero = jnp.zeros((), jnp.float32)
    for j in range(_NBUF):
        @pl.when(i > 0)
        def _():
            pltpu.make_async_copy(buf.at[j], o_ref.at[pl.ds(0, _SUB)],
                                  sems.at[j]).wait()

        xc = x_ref[0, 0, pl.ds(j * _SUB, _SUB)][:, None]
        buf[j] = jnp.where(xc == 1, r1, jnp.where(xc == 2, r2, zero))
        pltpu.async_copy(buf.at[j],
                         o_ref.at[pl.ds(i * _STEP + j * _SUB, _SUB)],
                         sems.at[j])

    @pl.when(i == pl.num_programs(0) - 1)
    def _():
        for j in range(_NBUF):
            pltpu.make_async_copy(buf.at[j], o_ref.at[pl.ds(0, _SUB)],
                                  sems.at[j]).wait()


def kernel(x, table):
    b, s = x.shape
    n = b * s
    grid = n // _STEP
    x_r = x.reshape(grid, 1, _STEP).astype(jnp.int32)
    out = pl.pallas_call(
        _emb_body,
        grid=(grid,),
        in_specs=[
            pl.BlockSpec((1, 1, _STEP), lambda i: (i, 0, 0)),
            pl.BlockSpec((_NUM_EMB, _HIDDEN), lambda i: (0, 0)),
        ],
        out_specs=pl.BlockSpec(memory_space=pl.ANY),
        out_shape=jax.ShapeDtypeStruct((n, _HIDDEN), jnp.float32),
        scratch_shapes=[
            pltpu.VMEM((_NBUF, _SUB, _HIDDEN), jnp.float32),
            pltpu.SemaphoreType.DMA((_NBUF,)),
        ],
    )(x_r, table)
    return out.reshape(b, s, _HIDDEN)


# speedup vs baseline: 1.0395x; 1.0395x over previous
import jax
import jax.numpy as jnp
from jax.experimental import pallas as pl

_HIDDEN = 1024
_NUM_EMB = 3
_CHUNK = 1024


def _emb_body(x_ref, t_ref, o_ref):
    xc = x_ref[0, 0, :][:, None]
    r1 = t_ref[1, :][None, :]
    r2 = t_ref[2, :][None, :]
    zero = jnp.zeros((), jnp.float32)
    o_ref[...] = jnp.where(xc == 1, r1, jnp.where(xc == 2, r2, zero))


def kernel(x, table):
    b, s = x.shape
    n = b * s
    grid = n // _CHUNK
    x_r = x.reshape(grid, 1, _CHUNK).astype(jnp.int32)
    out = pl.pallas_call(
        _emb_body,
        grid=(grid,),
        in_specs=[
            pl.BlockSpec((1, 1, _CHUNK), lambda i: (i, 0, 0)),
            pl.BlockSpec((_NUM_EMB, _HIDDEN), lambda i: (0, 0)),
        ],
        out_specs=pl.BlockSpec((_CHUNK, _HIDDEN), lambda i: (i, 0)),
        out_shape=jax.ShapeDtypeStruct((n, _HIDDEN), jnp.float32),
    )(x_r, table)
    return out.reshape(b, s, _HIDDEN)
